# baseline (speedup 1.0000x reference)
import functools

import jax
import jax.numpy as jnp
from jax import lax
from jax.experimental import pallas as pl
from jax.experimental.pallas import tpu as pltpu
from jax.experimental.pallas import tpu_sc as plsc

H = 128
G = 64

NCORES = 2
NSUB = 16
NW = NCORES * NSUB
GB = 80
LNEPS = 1e-5


def _node_block(n):
    return 2000 if n % 2000 == 0 else 8


def _sc_mesh():
    return plsc.VectorSubcoreMesh(core_axis_name="c", subcore_axis_name="s",
                                  num_cores=NCORES, num_subcores=NSUB)


def _edge_nn_body(he_ref, w1_ref, b1_ref, w2_ref, b2_ref, w_ref):
    x = he_ref[...]
    y = jnp.maximum(jnp.dot(x, w1_ref[...], preferred_element_type=jnp.float32)
                    + b1_ref[...], 0.0)
    z = jnp.maximum(jnp.dot(y, w2_ref[...], preferred_element_type=jnp.float32)
                    + b2_ref[...], 0.0)
    w_ref[...] = z.astype(jnp.bfloat16)


def _edge_nn(h_E, W1, b1, W2, b2, blk):
    E = h_E.shape[0]
    grid = E // blk
    return pl.pallas_call(
        _edge_nn_body,
        grid=(grid,),
        in_specs=[
            pl.BlockSpec((blk, H), lambda i: (i, 0)),
            pl.BlockSpec((H, H), lambda i: (0, 0)),
            pl.BlockSpec((1, H), lambda i: (0, 0)),
            pl.BlockSpec((H, H), lambda i: (0, 0)),
            pl.BlockSpec((1, H), lambda i: (0, 0)),
        ],
        out_specs=pl.BlockSpec((blk, H), lambda i: (i, 0)),
        out_shape=jax.ShapeDtypeStruct((E, H), jnp.bfloat16),
    )(h_E, W1, b1, W2, b2)


def _mul_body(xg_ref, w_ref, o_ref):
    o_ref[...] = xg_ref[...].astype(jnp.float32) * w_ref[...].astype(jnp.float32)


def _mul(xg, w, blk):
    E = xg.shape[0]
    grid = E // blk
    return pl.pallas_call(
        _mul_body,
        grid=(grid,),
        in_specs=[
            pl.BlockSpec((blk, H), lambda i: (i, 0)),
            pl.BlockSpec((blk, H), lambda i: (i, 0)),
        ],
        out_specs=pl.BlockSpec((blk, H), lambda i: (i, 0)),
        out_shape=jax.ShapeDtypeStruct((E, H), jnp.float32),
    )(xg, w)


def _node_update_body(hv_ref, x_ref, agg_ref, wr_ref, br_ref,
                      g_ref, be_ref, o_ref):
    y = (hv_ref[...] + agg_ref[...] + br_ref[...]
         + jnp.dot(x_ref[...], wr_ref[...], preferred_element_type=jnp.float32))
    m = jnp.mean(y, axis=1, keepdims=True)
    c = y - m
    v = jnp.mean(c * c, axis=1, keepdims=True)
    o_ref[...] = c * lax.rsqrt(v + LNEPS) * g_ref[...] + be_ref[...]


def _node_update(h_V, x, agg, Wr, br, g, be):
    N = h_V.shape[0]
    blk = _node_block(N)
    grid = N // blk
    return pl.pallas_call(
        _node_update_body,
        grid=(grid,),
        in_specs=[
            pl.BlockSpec((blk, H), lambda i: (i, 0)),
            pl.BlockSpec((blk, H), lambda i: (i, 0)),
            pl.BlockSpec((blk, H), lambda i: (i, 0)),
            pl.BlockSpec((H, H), lambda i: (0, 0)),
            pl.BlockSpec((1, H), lambda i: (0, 0)),
            pl.BlockSpec((1, H), lambda i: (0, 0)),
            pl.BlockSpec((1, H), lambda i: (0, 0)),
        ],
        out_specs=pl.BlockSpec((blk, H), lambda i: (i, 0)),
        out_shape=jax.ShapeDtypeStruct((N, H), jnp.float32),
    )(h_V, x, agg, Wr, br, g, be)


def _node_update2_body(hv_ref, x_ref, agg_ref, wr_ref, br_ref,
                       g_ref, be_ref, wa_ref, wc_ref, o_ref, s_ref, d_ref):
    y = (hv_ref[...] + agg_ref[...] + br_ref[...]
         + jnp.dot(x_ref[...], wr_ref[...], preferred_element_type=jnp.float32))
    m = jnp.mean(y, axis=1, keepdims=True)
    c = y - m
    v = jnp.mean(c * c, axis=1, keepdims=True)
    h2 = c * lax.rsqrt(v + LNEPS) * g_ref[...] + be_ref[...]
    o_ref[...] = h2
    s_ref[...] = jnp.dot(h2, wa_ref[...], preferred_element_type=jnp.float32)
    d_ref[...] = jnp.dot(h2, wc_ref[...], preferred_element_type=jnp.float32)


def _node_update2(h_V, x, agg, Wr, br, g, be, W11a, W11c):
    N = h_V.shape[0]
    blk = _node_block(N)
    grid = N // blk
    return pl.pallas_call(
        _node_update2_body,
        grid=(grid,),
        in_specs=[
            pl.BlockSpec((blk, H), lambda i: (i, 0)),
            pl.BlockSpec((blk, H), lambda i: (i, 0)),
            pl.BlockSpec((blk, H), lambda i: (i, 0)),
            pl.BlockSpec((H, H), lambda i: (0, 0)),
            pl.BlockSpec((1, H), lambda i: (0, 0)),
            pl.BlockSpec((1, H), lambda i: (0, 0)),
            pl.BlockSpec((1, H), lambda i: (0, 0)),
            pl.BlockSpec((H, H), lambda i: (0, 0)),
            pl.BlockSpec((H, H), lambda i: (0, 0)),
        ],
        out_specs=[
            pl.BlockSpec((blk, H), lambda i: (i, 0)),
            pl.BlockSpec((blk, H), lambda i: (i, 0)),
            pl.BlockSpec((blk, H), lambda i: (i, 0)),
        ],
        out_shape=[
            jax.ShapeDtypeStruct((N, H), jnp.float32),
            jax.ShapeDtypeStruct((N, H), jnp.float32),
            jax.ShapeDtypeStruct((N, H), jnp.float32),
        ],
    )(h_V, x, agg, Wr, br, g, be, W11a, W11c)


def _edge_mlp_body(sg_ref, dg_ref, he_ref, wb_ref, b11_ref, w12_ref, b12_ref,
                   gbn_ref, bbn_ref, o_ref):
    he = he_ref[...]
    pre = (sg_ref[...] + dg_ref[...] + b11_ref[...]
           + jnp.dot(he, wb_ref[...], preferred_element_type=jnp.float32))
    act = 0.5 * pre * (1.0 + lax.erf(pre * (1.0 / jnp.sqrt(2.0).astype(jnp.float32))))
    msg = jnp.dot(act, w12_ref[...], preferred_element_type=jnp.float32) + b12_ref[...]
    scale = gbn_ref[...] * (1.0 / jnp.sqrt(jnp.float32(1.0 + LNEPS)))
    o_ref[...] = (he + msg) * scale + bbn_ref[...]


def _edge_mlp(sg, dg, h_E, W11b, b11, W12, b12, gbn, bbn, blk):
    E = h_E.shape[0]
    grid = E // blk
    return pl.pallas_call(
        _edge_mlp_body,
        grid=(grid,),
        in_specs=[
            pl.BlockSpec((blk, H), lambda i: (i, 0)),
            pl.BlockSpec((blk, H), lambda i: (i, 0)),
            pl.BlockSpec((blk, H), lambda i: (i, 0)),
            pl.BlockSpec((H, H), lambda i: (0, 0)),
            pl.BlockSpec((1, H), lambda i: (0, 0)),
            pl.BlockSpec((H, H), lambda i: (0, 0)),
            pl.BlockSpec((1, H), lambda i: (0, 0)),
            pl.BlockSpec((1, H), lambda i: (0, 0)),
            pl.BlockSpec((1, H), lambda i: (0, 0)),
        ],
        out_specs=pl.BlockSpec((blk, H), lambda i: (i, 0)),
        out_shape=jax.ShapeDtypeStruct((E, H), jnp.float32),
    )(sg, dg, h_E, W11b, b11, W12, b12, gbn, bbn)


def _context_gate_body(h2_ref, bid_ref, wg1_ref, bg1_ref, wg2_ref, bg2_ref,
                       gate_ref, csum_ref, cnt_ref):
    i = pl.program_id(0)
    nblk = pl.num_programs(0)

    @pl.when(i == 0)
    def _():
        csum_ref[...] = jnp.zeros_like(csum_ref)
        cnt_ref[...] = jnp.zeros_like(cnt_ref)

    b = bid_ref[0]
    gi = lax.broadcasted_iota(jnp.int32, (G, b.shape[1]), 0)
    mask = (b == gi).astype(jnp.float32)
    csum_ref[...] += jnp.dot(mask, h2_ref[...], preferred_element_type=jnp.float32)
    cnt_ref[...] += jnp.broadcast_to(jnp.sum(mask, axis=1, keepdims=True),
                                     cnt_ref.shape)

    @pl.when(i == nblk - 1)
    def _():
        c = csum_ref[...] / jnp.maximum(cnt_ref[...], 1.0)
        t = jnp.maximum(jnp.dot(c, wg1_ref[...], preferred_element_type=jnp.float32)
                        + bg1_ref[...], 0.0)
        t = jnp.dot(t, wg2_ref[...], preferred_element_type=jnp.float32) + bg2_ref[...]
        gate_ref[...] = 1.0 / (1.0 + jnp.exp(-t))


def _context_gate(h2, bid3, Wg1, bg1, Wg2, bg2):
    N = h2.shape[0]
    blk = _node_block(N)
    grid = N // blk
    return pl.pallas_call(
        _context_gate_body,
        grid=(grid,),
        in_specs=[
            pl.BlockSpec((blk, H), lambda i: (i, 0)),
            pl.BlockSpec((1, 1, blk), lambda i: (i, 0, 0)),
            pl.BlockSpec((H, H), lambda i: (0, 0)),
            pl.BlockSpec((1, H), lambda i: (0, 0)),
            pl.BlockSpec((H, H), lambda i: (0, 0)),
            pl.BlockSpec((1, H), lambda i: (0, 0)),
        ],
        out_specs=pl.BlockSpec((G, H), lambda i: (0, 0)),
        out_shape=jax.ShapeDtypeStruct((G, H), jnp.float32),
        scratch_shapes=[
            pltpu.VMEM((G, H), jnp.float32),
            pltpu.VMEM((G, H), jnp.float32),
        ],
    )(h2, bid3, Wg1, bg1, Wg2, bg2)


def _apply_gate_body(h2_ref, bid_ref, gate_ref, o_ref):
    b = bid_ref[0]
    blk = b.shape[1]
    bt = jnp.reshape(b, (blk, 1))
    gi = lax.broadcasted_iota(jnp.int32, (blk, G), 1)
    mask = (bt == gi).astype(jnp.float32)
    g = jnp.dot(mask, gate_ref[...], preferred_element_type=jnp.float32)
    o_ref[...] = h2_ref[...] * g


def _apply_gate(h2, bid3, gate):
    N = h2.shape[0]
    blk = _node_block(N)
    grid = N // blk
    return pl.pallas_call(
        _apply_gate_body,
        grid=(grid,),
        in_specs=[
            pl.BlockSpec((blk, H), lambda i: (i, 0)),
            pl.BlockSpec((1, 1, blk), lambda i: (i, 0, 0)),
            pl.BlockSpec((G, H), lambda i: (0, 0)),
        ],
        out_specs=pl.BlockSpec((blk, H), lambda i: (i, 0)),
        out_shape=jax.ShapeDtypeStruct((N, H), jnp.float32),
    )(h2, bid3, gate)


def _gather_one(x, idx3, E):
    ngrp = idx3.shape[1]
    ept = ngrp * GB
    W = x.shape[1]

    @functools.partial(
        pl.kernel,
        out_type=jax.ShapeDtypeStruct((E, W), x.dtype),
        mesh=_sc_mesh(),
        scratch_types=[
            pltpu.VMEM((ngrp, GB), jnp.int32),
            pltpu.VMEM((GB, W), x.dtype),
            pltpu.VMEM((GB, W), x.dtype),
            pltpu.VMEM((GB, W), x.dtype),
            pltpu.VMEM((GB, W), x.dtype),
            pltpu.SemaphoreType.DMA,
            pltpu.SemaphoreType.DMA,
            pltpu.SemaphoreType.DMA,
            pltpu.SemaphoreType.DMA,
            pltpu.SemaphoreType.DMA,
        ],
    )
    def k(x_hbm, idx_hbm, out_hbm, sidx, ra, rb, rc, rd,
          fsa, fsb, fsc, fsd, wsem):
        cid = lax.axis_index("c")
        sid = lax.axis_index("s")
        tid = cid * NSUB + sid

        pltpu.sync_copy(idx_hbm.at[tid], sidx)
        base_e = tid * ept

        def body4(i, _):
            g0 = 4 * i
            fa = pltpu.async_copy(x_hbm.at[sidx.at[g0]], ra, fsa)
            fb = pltpu.async_copy(x_hbm.at[sidx.at[g0 + 1]], rb, fsb)
            fc = pltpu.async_copy(x_hbm.at[sidx.at[g0 + 2]], rc, fsc)
            fd = pltpu.async_copy(x_hbm.at[sidx.at[g0 + 3]], rd, fsd)
            fa.wait()
            wa = pltpu.async_copy(ra, out_hbm.at[pl.ds(base_e + g0 * GB, GB)],
                                  wsem)
            fb.wait()
            wb = pltpu.async_copy(
                rb, out_hbm.at[pl.ds(base_e + (g0 + 1) * GB, GB)], wsem)
            fc.wait()
            wc = pltpu.async_copy(
                rc, out_hbm.at[pl.ds(base_e + (g0 + 2) * GB, GB)], wsem)
            fd.wait()
            wd = pltpu.async_copy(
                rd, out_hbm.at[pl.ds(base_e + (g0 + 3) * GB, GB)], wsem)
            wa.wait()
            wb.wait()
            wc.wait()
            wd.wait()
            return 0
        lax.fori_loop(0, ngrp // 4, body4, 0)

        for g in range(ngrp - ngrp % 4, ngrp):
            pltpu.async_copy(x_hbm.at[sidx.at[g]], ra, fsa).wait()
            pltpu.sync_copy(ra, out_hbm.at[pl.ds(base_e + g * GB, GB)])

    return k(x, idx3)


def _scatter_add(msg, dst3, nh):
    ngrp = dst3.shape[1]
    ept = ngrp * GB
    stripe = nh // NSUB
    zrep = stripe // GB

    @functools.partial(
        pl.kernel,
        out_type=jax.ShapeDtypeStruct((NCORES, nh, H), jnp.float32),
        mesh=_sc_mesh(),
        scratch_types=[
            pltpu.VMEM((ngrp, GB), jnp.int32),
            pltpu.VMEM((GB, H), jnp.float32),
            pltpu.VMEM((GB, H), jnp.float32),
            pltpu.VMEM((GB, H), jnp.float32),
            pltpu.VMEM((GB, H), jnp.float32),
            pltpu.VMEM_SHARED((nh + GB, H), jnp.float32),
            pltpu.SemaphoreType.DMA,
            pltpu.SemaphoreType.DMA,
            pltpu.SemaphoreType.DMA,
            pltpu.SemaphoreType.DMA,
            pltpu.SemaphoreType.DMA,
        ],
    )
    def k(msg_hbm, dst_hbm, out_hbm, didx, ra, rb, rc, rd, aggr,
          fsa, fsb, fsc, fsd, ssem):
        cid = lax.axis_index("c")
        sid = lax.axis_index("s")

        def zb(i, _):
            ra[i // 8, pl.ds((i % 8) * 16, 16)] = jnp.zeros((16,), jnp.float32)
            return 0
        lax.fori_loop(0, GB * 8, zb, 0)

        def zcopy(j, _):
            pltpu.sync_copy(ra, aggr.at[pl.ds(sid * stripe + j * GB, GB)])
            return 0
        lax.fori_loop(0, zrep, zcopy, 0)
        plsc.subcore_barrier()

        pltpu.sync_copy(dst_hbm.at[sid], didx)

        base = cid * nh

        def adj(i, _):
            for cc in range(GB // 16):
                sl = pl.ds(cc * 16, 16)
                dv = didx[i, sl] - base
                ok = (dv >= 0) & (dv < nh)
                didx[i, sl] = jnp.where(ok, dv, nh)
            return 0
        lax.fori_loop(0, ngrp, adj, 0)

        base_e = sid * ept

        def body4(i, _):
            g0 = 4 * i
            fa = pltpu.async_copy(msg_hbm.at[pl.ds(base_e + g0 * GB, GB)],
                                  ra, fsa)
            fb = pltpu.async_copy(msg_hbm.at[pl.ds(base_e + (g0 + 1) * GB, GB)],
                                  rb, fsb)
            fc = pltpu.async_copy(msg_hbm.at[pl.ds(base_e + (g0 + 2) * GB, GB)],
                                  rc, fsc)
            fd = pltpu.async_copy(msg_hbm.at[pl.ds(base_e + (g0 + 3) * GB, GB)],
                                  rd, fsd)
            fa.wait()
            sa = pltpu.async_copy(ra, aggr.at[didx.at[g0]], ssem, add=True)
            fb.wait()
            sb = pltpu.async_copy(rb, aggr.at[didx.at[g0 + 1]], ssem, add=True)
            fc.wait()
            sc = pltpu.async_copy(rc, aggr.at[didx.at[g0 + 2]], ssem, add=True)
            fd.wait()
            sd = pltpu.async_copy(rd, aggr.at[didx.at[g0 + 3]], ssem, add=True)
            sa.wait()
            sb.wait()
            sc.wait()
            sd.wait()
            return 0
        lax.fori_loop(0, ngrp // 4, body4, 0)

        for g in range(ngrp - ngrp % 4, ngrp):
            pltpu.sync_copy(msg_hbm.at[pl.ds(base_e + g * GB, GB)], ra)
            pltpu.sync_copy(ra, aggr.at[didx.at[g]], add=True)
        plsc.subcore_barrier()

        def wout(j, _):
            sl = pl.ds(sid * stripe + j * GB, GB)
            pltpu.sync_copy(aggr.at[sl], out_hbm.at[cid].at[sl])
            return 0
        lax.fori_loop(0, zrep, wout, 0)

    return k(msg, dst3)


def _gather_pair(s, d, src3, dst3, E):
    ngrp = src3.shape[1]
    ept = ngrp * GB
    W = s.shape[1]

    @functools.partial(
        pl.kernel,
        out_type=[
            jax.ShapeDtypeStruct((E, W), s.dtype),
            jax.ShapeDtypeStruct((E, W), s.dtype),
        ],
        mesh=_sc_mesh(),
        scratch_types=[
            pltpu.VMEM((ngrp, GB), jnp.int32),
            pltpu.VMEM((ngrp, GB), jnp.int32),
            pltpu.VMEM((GB, W), s.dtype),
            pltpu.VMEM((GB, W), s.dtype),
            pltpu.VMEM((GB, W), s.dtype),
            pltpu.VMEM((GB, W), s.dtype),
            pltpu.SemaphoreType.DMA,
            pltpu.SemaphoreType.DMA,
            pltpu.SemaphoreType.DMA,
            pltpu.SemaphoreType.DMA,
            pltpu.SemaphoreType.DMA,
        ],
    )
    def k(s_hbm, d_hbm, src_hbm, dst_hbm, sg_hbm, dg_hbm, sidx, didx,
          sa, da, sb, db, fs1, fs2, fs3, fs4, wsem):
        cid = lax.axis_index("c")
        sid = lax.axis_index("s")
        tid = cid * NSUB + sid

        pltpu.sync_copy(src_hbm.at[tid], sidx)
        pltpu.sync_copy(dst_hbm.at[tid], didx)
        base_e = tid * ept

        def body2(i, _):
            g0 = 2 * i
            fsa = pltpu.async_copy(s_hbm.at[sidx.at[g0]], sa, fs1)
            fda = pltpu.async_copy(d_hbm.at[didx.at[g0]], da, fs2)
            fsb = pltpu.async_copy(s_hbm.at[sidx.at[g0 + 1]], sb, fs3)
            fdb = pltpu.async_copy(d_hbm.at[didx.at[g0 + 1]], db, fs4)
            fsa.wait()
            w1 = pltpu.async_copy(sa, sg_hbm.at[pl.ds(base_e + g0 * GB, GB)],
                                  wsem)
            fda.wait()
            w2 = pltpu.async_copy(da, dg_hbm.at[pl.ds(base_e + g0 * GB, GB)],
                                  wsem)
            fsb.wait()
            w3 = pltpu.async_copy(
                sb, sg_hbm.at[pl.ds(base_e + (g0 + 1) * GB, GB)], wsem)
            fdb.wait()
            w4 = pltpu.async_copy(
                db, dg_hbm.at[pl.ds(base_e + (g0 + 1) * GB, GB)], wsem)
            w1.wait()
            w2.wait()
            w3.wait()
            w4.wait()
            return 0
        lax.fori_loop(0, ngrp // 2, body2, 0)

        g = ngrp - 1
        cps = pltpu.async_copy(s_hbm.at[sidx.at[g]], sa, fs1)
        cpd = pltpu.async_copy(d_hbm.at[didx.at[g]], da, fs2)
        cps.wait()
        pltpu.sync_copy(sa, sg_hbm.at[pl.ds(base_e + g * GB, GB)])
        cpd.wait()
        pltpu.sync_copy(da, dg_hbm.at[pl.ds(base_e + g * GB, GB)])

    return k(s, d, src3, dst3)


def kernel(h_V, edge_index, h_E, batch_id, W1, b1, W2, b2, Wr, br, W11, b11,
           W12, b12, Wg1, bg1, Wg2, bg2, g0, be0, g1, be1, gbn, bbn):
    N = h_V.shape[0]
    E = h_E.shape[0]
    blk_e = 2000
    nblk = _node_block(N)
    nh = -(-N // (2 * NSUB * GB)) * GB * NSUB

    src_w = edge_index[0].reshape(NW, E // (NW * GB), GB)
    dst_w = edge_index[1].reshape(NW, E // (NW * GB), GB)
    dst_s = edge_index[1].reshape(NSUB, E // (NSUB * GB), GB)
    bid3 = batch_id.reshape(N // nblk, 1, nblk)

    r = lambda v: v.reshape(1, H)
    b1r, b2r, brr, b11r, b12r = r(b1), r(b2), r(br), r(b11), r(b12)
    bg1r, bg2r = r(bg1), r(bg2)
    g0r, be0r, g1r, be1r, gbnr, bbnr = r(g0), r(be0), r(g1), r(be1), r(gbn), r(bbn)
    W11a = W11[0:H]
    W11b = W11[H:2 * H]
    W11c = W11[2 * H:3 * H]

    xg1 = _gather_one(h_V, src_w, E)
    w = _edge_nn(h_E, W1, b1r, W2, b2r, blk_e)
    msg1 = _mul(xg1, w, blk_e)
    agg1 = _scatter_add(msg1, dst_s, nh).reshape(2 * nh, H)
    h1 = _node_update(h_V, h_V, agg1[:N], Wr, brr, g0r, be0r)

    xg2 = _gather_one(h1, src_w, E)
    msg2 = _mul(xg2, w, blk_e)
    agg2 = _scatter_add(msg2, dst_s, nh).reshape(2 * nh, H)
    h2, s, d = _node_update2(h_V, h1, agg2[:N], Wr, brr, g1r, be1r, W11a, W11c)

    sg, dg = _gather_pair(s, d, src_w, dst_w, E)
    h_E_out = _edge_mlp(sg, dg, h_E, W11b, b11r, W12, b12r, gbnr, bbnr, blk_e)

    gate = _context_gate(h2, bid3, Wg1, bg1r, Wg2, bg2r)
    h_V_out = _apply_gate(h2, bid3, gate)

    return (h_V_out, h_E_out)


# speedup vs baseline: 1.0270x; 1.0270x over previous
import functools

import jax
import jax.numpy as jnp
from jax import lax
from jax.experimental import pallas as pl
from jax.experimental.pallas import tpu as pltpu
from jax.experimental.pallas import tpu_sc as plsc

H = 128
G = 64

NCORES = 2
NSUB = 16
NW = NCORES * NSUB
GB = 80
LNEPS = 1e-5


def _node_block(n):
    return 2000 if n % 2000 == 0 else 8


def _sc_mesh():
    return plsc.VectorSubcoreMesh(core_axis_name="c", subcore_axis_name="s",
                                  num_cores=NCORES, num_subcores=NSUB)


def _edge_nn_mul_body(he_ref, xg_ref, w1_ref, b1_ref, w2_ref, b2_ref,
                      w_ref, msg_ref):
    x = he_ref[...]
    y = jnp.maximum(jnp.dot(x, w1_ref[...], preferred_element_type=jnp.float32)
                    + b1_ref[...], 0.0)
    z = jnp.maximum(jnp.dot(y, w2_ref[...], preferred_element_type=jnp.float32)
                    + b2_ref[...], 0.0)
    w_ref[...] = z.astype(jnp.bfloat16)
    msg_ref[...] = xg_ref[...].astype(jnp.float32) * z


def _edge_nn_mul(h_E, xg, W1, b1, W2, b2, blk):
    E = h_E.shape[0]
    grid = E // blk
    return pl.pallas_call(
        _edge_nn_mul_body,
        grid=(grid,),
        in_specs=[
            pl.BlockSpec((blk, H), lambda i: (i, 0)),
            pl.BlockSpec((blk, H), lambda i: (i, 0)),
            pl.BlockSpec((H, H), lambda i: (0, 0)),
            pl.BlockSpec((1, H), lambda i: (0, 0)),
            pl.BlockSpec((H, H), lambda i: (0, 0)),
            pl.BlockSpec((1, H), lambda i: (0, 0)),
        ],
        out_specs=[
            pl.BlockSpec((blk, H), lambda i: (i, 0)),
            pl.BlockSpec((blk, H), lambda i: (i, 0)),
        ],
        out_shape=[
            jax.ShapeDtypeStruct((E, H), jnp.bfloat16),
            jax.ShapeDtypeStruct((E, H), jnp.float32),
        ],
    )(h_E, xg, W1, b1, W2, b2)


def _mul_body(xg_ref, w_ref, o_ref):
    o_ref[...] = xg_ref[...].astype(jnp.float32) * w_ref[...].astype(jnp.float32)


def _mul(xg, w, blk):
    E = xg.shape[0]
    grid = E // blk
    return pl.pallas_call(
        _mul_body,
        grid=(grid,),
        in_specs=[
            pl.BlockSpec((blk, H), lambda i: (i, 0)),
            pl.BlockSpec((blk, H), lambda i: (i, 0)),
        ],
        out_specs=pl.BlockSpec((blk, H), lambda i: (i, 0)),
        out_shape=jax.ShapeDtypeStruct((E, H), jnp.float32),
    )(xg, w)


def _node_update_body(hv_ref, x_ref, agg_ref, wr_ref, br_ref,
                      g_ref, be_ref, o_ref):
    y = (hv_ref[...] + agg_ref[...] + br_ref[...]
         + jnp.dot(x_ref[...], wr_ref[...], preferred_element_type=jnp.float32))
    m = jnp.mean(y, axis=1, keepdims=True)
    c = y - m
    v = jnp.mean(c * c, axis=1, keepdims=True)
    o_ref[...] = c * lax.rsqrt(v + LNEPS) * g_ref[...] + be_ref[...]


def _node_update(h_V, x, agg, Wr, br, g, be):
    N = h_V.shape[0]
    blk = _node_block(N)
    grid = N // blk
    return pl.pallas_call(
        _node_update_body,
        grid=(grid,),
        in_specs=[
            pl.BlockSpec((blk, H), lambda i: (i, 0)),
            pl.BlockSpec((blk, H), lambda i: (i, 0)),
            pl.BlockSpec((blk, H), lambda i: (i, 0)),
            pl.BlockSpec((H, H), lambda i: (0, 0)),
            pl.BlockSpec((1, H), lambda i: (0, 0)),
            pl.BlockSpec((1, H), lambda i: (0, 0)),
            pl.BlockSpec((1, H), lambda i: (0, 0)),
        ],
        out_specs=pl.BlockSpec((blk, H), lambda i: (i, 0)),
        out_shape=jax.ShapeDtypeStruct((N, H), jnp.float32),
    )(h_V, x, agg, Wr, br, g, be)


def _node_update2_body(hv_ref, x_ref, agg_ref, wr_ref, br_ref,
                       g_ref, be_ref, wa_ref, wc_ref, o_ref, s_ref, d_ref):
    y = (hv_ref[...] + agg_ref[...] + br_ref[...]
         + jnp.dot(x_ref[...], wr_ref[...], preferred_element_type=jnp.float32))
    m = jnp.mean(y, axis=1, keepdims=True)
    c = y - m
    v = jnp.mean(c * c, axis=1, keepdims=True)
    h2 = c * lax.rsqrt(v + LNEPS) * g_ref[...] + be_ref[...]
    o_ref[...] = h2
    s_ref[...] = jnp.dot(h2, wa_ref[...], preferred_element_type=jnp.float32)
    d_ref[...] = jnp.dot(h2, wc_ref[...], preferred_element_type=jnp.float32)


def _node_update2(h_V, x, agg, Wr, br, g, be, W11a, W11c):
    N = h_V.shape[0]
    blk = _node_block(N)
    grid = N // blk
    return pl.pallas_call(
        _node_update2_body,
        grid=(grid,),
        in_specs=[
            pl.BlockSpec((blk, H), lambda i: (i, 0)),
            pl.BlockSpec((blk, H), lambda i: (i, 0)),
            pl.BlockSpec((blk, H), lambda i: (i, 0)),
            pl.BlockSpec((H, H), lambda i: (0, 0)),
            pl.BlockSpec((1, H), lambda i: (0, 0)),
            pl.BlockSpec((1, H), lambda i: (0, 0)),
            pl.BlockSpec((1, H), lambda i: (0, 0)),
            pl.BlockSpec((H, H), lambda i: (0, 0)),
            pl.BlockSpec((H, H), lambda i: (0, 0)),
        ],
        out_specs=[
            pl.BlockSpec((blk, H), lambda i: (i, 0)),
            pl.BlockSpec((blk, H), lambda i: (i, 0)),
            pl.BlockSpec((blk, H), lambda i: (i, 0)),
        ],
        out_shape=[
            jax.ShapeDtypeStruct((N, H), jnp.float32),
            jax.ShapeDtypeStruct((N, H), jnp.float32),
            jax.ShapeDtypeStruct((N, H), jnp.float32),
        ],
    )(h_V, x, agg, Wr, br, g, be, W11a, W11c)


def _edge_mlp_body(sg_ref, dg_ref, he_ref, wb_ref, b11_ref, w12_ref, b12_ref,
                   gbn_ref, bbn_ref, o_ref):
    he = he_ref[...]
    pre = (sg_ref[...] + dg_ref[...] + b11_ref[...]
           + jnp.dot(he, wb_ref[...], preferred_element_type=jnp.float32))
    act = 0.5 * pre * (1.0 + lax.erf(pre * (1.0 / jnp.sqrt(2.0).astype(jnp.float32))))
    msg = jnp.dot(act, w12_ref[...], preferred_element_type=jnp.float32) + b12_ref[...]
    scale = gbn_ref[...] * (1.0 / jnp.sqrt(jnp.float32(1.0 + LNEPS)))
    o_ref[...] = (he + msg) * scale + bbn_ref[...]


def _edge_mlp(sg, dg, h_E, W11b, b11, W12, b12, gbn, bbn, blk):
    E = h_E.shape[0]
    grid = E // blk
    return pl.pallas_call(
        _edge_mlp_body,
        grid=(grid,),
        in_specs=[
            pl.BlockSpec((blk, H), lambda i: (i, 0)),
            pl.BlockSpec((blk, H), lambda i: (i, 0)),
            pl.BlockSpec((blk, H), lambda i: (i, 0)),
            pl.BlockSpec((H, H), lambda i: (0, 0)),
            pl.BlockSpec((1, H), lambda i: (0, 0)),
            pl.BlockSpec((H, H), lambda i: (0, 0)),
            pl.BlockSpec((1, H), lambda i: (0, 0)),
            pl.BlockSpec((1, H), lambda i: (0, 0)),
            pl.BlockSpec((1, H), lambda i: (0, 0)),
        ],
        out_specs=pl.BlockSpec((blk, H), lambda i: (i, 0)),
        out_shape=jax.ShapeDtypeStruct((E, H), jnp.float32),
    )(sg, dg, h_E, W11b, b11, W12, b12, gbn, bbn)


def _context_gate_body(h2_ref, bid_ref, wg1_ref, bg1_ref, wg2_ref, bg2_ref,
                       gate_ref, csum_ref, cnt_ref):
    i = pl.program_id(0)
    nblk = pl.num_programs(0)

    @pl.when(i == 0)
    def _():
        csum_ref[...] = jnp.zeros_like(csum_ref)
        cnt_ref[...] = jnp.zeros_like(cnt_ref)

    b = bid_ref[0]
    gi = lax.broadcasted_iota(jnp.int32, (G, b.shape[1]), 0)
    mask = (b == gi).astype(jnp.float32)
    csum_ref[...] += jnp.dot(mask, h2_ref[...], preferred_element_type=jnp.float32)
    cnt_ref[...] += jnp.broadcast_to(jnp.sum(mask, axis=1, keepdims=True),
                                     cnt_ref.shape)

    @pl.when(i == nblk - 1)
    def _():
        c = csum_ref[...] / jnp.maximum(cnt_ref[...], 1.0)
        t = jnp.maximum(jnp.dot(c, wg1_ref[...], preferred_element_type=jnp.float32)
                        + bg1_ref[...], 0.0)
        t = jnp.dot(t, wg2_ref[...], preferred_element_type=jnp.float32) + bg2_ref[...]
        gate_ref[...] = 1.0 / (1.0 + jnp.exp(-t))


def _context_gate(h2, bid3, Wg1, bg1, Wg2, bg2):
    N = h2.shape[0]
    blk = _node_block(N)
    grid = N // blk
    return pl.pallas_call(
        _context_gate_body,
        grid=(grid,),
        in_specs=[
            pl.BlockSpec((blk, H), lambda i: (i, 0)),
            pl.BlockSpec((1, 1, blk), lambda i: (i, 0, 0)),
            pl.BlockSpec((H, H), lambda i: (0, 0)),
            pl.BlockSpec((1, H), lambda i: (0, 0)),
            pl.BlockSpec((H, H), lambda i: (0, 0)),
            pl.BlockSpec((1, H), lambda i: (0, 0)),
        ],
        out_specs=pl.BlockSpec((G, H), lambda i: (0, 0)),
        out_shape=jax.ShapeDtypeStruct((G, H), jnp.float32),
        scratch_shapes=[
            pltpu.VMEM((G, H), jnp.float32),
            pltpu.VMEM((G, H), jnp.float32),
        ],
    )(h2, bid3, Wg1, bg1, Wg2, bg2)


def _apply_gate_body(h2_ref, bid_ref, gate_ref, o_ref):
    b = bid_ref[0]
    blk = b.shape[1]
    bt = jnp.reshape(b, (blk, 1))
    gi = lax.broadcasted_iota(jnp.int32, (blk, G), 1)
    mask = (bt == gi).astype(jnp.float32)
    g = jnp.dot(mask, gate_ref[...], preferred_element_type=jnp.float32)
    o_ref[...] = h2_ref[...] * g


def _apply_gate(h2, bid3, gate):
    N = h2.shape[0]
    blk = _node_block(N)
    grid = N // blk
    return pl.pallas_call(
        _apply_gate_body,
        grid=(grid,),
        in_specs=[
            pl.BlockSpec((blk, H), lambda i: (i, 0)),
            pl.BlockSpec((1, 1, blk), lambda i: (i, 0, 0)),
            pl.BlockSpec((G, H), lambda i: (0, 0)),
        ],
        out_specs=pl.BlockSpec((blk, H), lambda i: (i, 0)),
        out_shape=jax.ShapeDtypeStruct((N, H), jnp.float32),
    )(h2, bid3, gate)


def _gather_one(x, idx3, E):
    ngrp = idx3.shape[1]
    ept = ngrp * GB
    W = x.shape[1]

    @functools.partial(
        pl.kernel,
        out_type=jax.ShapeDtypeStruct((E, W), x.dtype),
        mesh=_sc_mesh(),
        scratch_types=[
            pltpu.VMEM((ngrp, GB), jnp.int32),
            pltpu.VMEM((GB, W), x.dtype),
            pltpu.VMEM((GB, W), x.dtype),
            pltpu.VMEM((GB, W), x.dtype),
            pltpu.VMEM((GB, W), x.dtype),
            pltpu.SemaphoreType.DMA,
            pltpu.SemaphoreType.DMA,
            pltpu.SemaphoreType.DMA,
            pltpu.SemaphoreType.DMA,
            pltpu.SemaphoreType.DMA,
        ],
    )
    def k(x_hbm, idx_hbm, out_hbm, sidx, ra, rb, rc, rd,
          fsa, fsb, fsc, fsd, wsem):
        cid = lax.axis_index("c")
        sid = lax.axis_index("s")
        tid = cid * NSUB + sid

        pltpu.sync_copy(idx_hbm.at[tid], sidx)
        base_e = tid * ept

        def body4(i, _):
            g0 = 4 * i
            fa = pltpu.async_copy(x_hbm.at[sidx.at[g0]], ra, fsa)
            fb = pltpu.async_copy(x_hbm.at[sidx.at[g0 + 1]], rb, fsb)
            fc = pltpu.async_copy(x_hbm.at[sidx.at[g0 + 2]], rc, fsc)
            fd = pltpu.async_copy(x_hbm.at[sidx.at[g0 + 3]], rd, fsd)
            fa.wait()
            wa = pltpu.async_copy(ra, out_hbm.at[pl.ds(base_e + g0 * GB, GB)],
                                  wsem)
            fb.wait()
            wb = pltpu.async_copy(
                rb, out_hbm.at[pl.ds(base_e + (g0 + 1) * GB, GB)], wsem)
            fc.wait()
            wc = pltpu.async_copy(
                rc, out_hbm.at[pl.ds(base_e + (g0 + 2) * GB, GB)], wsem)
            fd.wait()
            wd = pltpu.async_copy(
                rd, out_hbm.at[pl.ds(base_e + (g0 + 3) * GB, GB)], wsem)
            wa.wait()
            wb.wait()
            wc.wait()
            wd.wait()
            return 0
        lax.fori_loop(0, ngrp // 4, body4, 0)

        for g in range(ngrp - ngrp % 4, ngrp):
            pltpu.async_copy(x_hbm.at[sidx.at[g]], ra, fsa).wait()
            pltpu.sync_copy(ra, out_hbm.at[pl.ds(base_e + g * GB, GB)])

    return k(x, idx3)


def _scatter_add(msg, dst3, nh):
    ngrp = dst3.shape[1]
    ept = ngrp * GB
    stripe = nh // NSUB
    zrep = stripe // GB

    @functools.partial(
        pl.kernel,
        out_type=jax.ShapeDtypeStruct((NCORES, nh, H), jnp.float32),
        mesh=_sc_mesh(),
        scratch_types=[
            pltpu.VMEM((ngrp, GB), jnp.int32),
            pltpu.VMEM((GB, H), jnp.float32),
            pltpu.VMEM((GB, H), jnp.float32),
            pltpu.VMEM((GB, H), jnp.float32),
            pltpu.VMEM((GB, H), jnp.float32),
            pltpu.VMEM_SHARED((nh + GB, H), jnp.float32),
            pltpu.SemaphoreType.DMA,
            pltpu.SemaphoreType.DMA,
            pltpu.SemaphoreType.DMA,
            pltpu.SemaphoreType.DMA,
            pltpu.SemaphoreType.DMA,
        ],
    )
    def k(msg_hbm, dst_hbm, out_hbm, didx, ra, rb, rc, rd, aggr,
          fsa, fsb, fsc, fsd, ssem):
        cid = lax.axis_index("c")
        sid = lax.axis_index("s")

        def zb(i, _):
            ra[i // 8, pl.ds((i % 8) * 16, 16)] = jnp.zeros((16,), jnp.float32)
            return 0
        lax.fori_loop(0, GB * 8, zb, 0)

        def zcopy(j, _):
            pltpu.sync_copy(ra, aggr.at[pl.ds(sid * stripe + j * GB, GB)])
            return 0
        lax.fori_loop(0, zrep, zcopy, 0)
        plsc.subcore_barrier()

        pltpu.sync_copy(dst_hbm.at[sid], didx)

        base = cid * nh

        def adj(i, _):
            for cc in range(GB // 16):
                sl = pl.ds(cc * 16, 16)
                dv = didx[i, sl] - base
                ok = (dv >= 0) & (dv < nh)
                didx[i, sl] = jnp.where(ok, dv, nh)
            return 0
        lax.fori_loop(0, ngrp, adj, 0)

        base_e = sid * ept

        def body4(i, _):
            g0 = 4 * i
            fa = pltpu.async_copy(msg_hbm.at[pl.ds(base_e + g0 * GB, GB)],
                                  ra, fsa)
            fb = pltpu.async_copy(msg_hbm.at[pl.ds(base_e + (g0 + 1) * GB, GB)],
                                  rb, fsb)
            fc = pltpu.async_copy(msg_hbm.at[pl.ds(base_e + (g0 + 2) * GB, GB)],
                                  rc, fsc)
            fd = pltpu.async_copy(msg_hbm.at[pl.ds(base_e + (g0 + 3) * GB, GB)],
                                  rd, fsd)
            fa.wait()
            sa = pltpu.async_copy(ra, aggr.at[didx.at[g0]], ssem, add=True)
            fb.wait()
            sb = pltpu.async_copy(rb, aggr.at[didx.at[g0 + 1]], ssem, add=True)
            fc.wait()
            sc = pltpu.async_copy(rc, aggr.at[didx.at[g0 + 2]], ssem, add=True)
            fd.wait()
            sd = pltpu.async_copy(rd, aggr.at[didx.at[g0 + 3]], ssem, add=True)
            sa.wait()
            sb.wait()
            sc.wait()
            sd.wait()
            return 0
        lax.fori_loop(0, ngrp // 4, body4, 0)

        for g in range(ngrp - ngrp % 4, ngrp):
            pltpu.sync_copy(msg_hbm.at[pl.ds(base_e + g * GB, GB)], ra)
            pltpu.sync_copy(ra, aggr.at[didx.at[g]], add=True)
        plsc.subcore_barrier()

        def wout(j, _):
            sl = pl.ds(sid * stripe + j * GB, GB)
            pltpu.sync_copy(aggr.at[sl], out_hbm.at[cid].at[sl])
            return 0
        lax.fori_loop(0, zrep, wout, 0)

    return k(msg, dst3)


def _gather_pair(s, d, src3, dst3, E):
    ngrp = src3.shape[1]
    ept = ngrp * GB
    W = s.shape[1]

    @functools.partial(
        pl.kernel,
        out_type=[
            jax.ShapeDtypeStruct((E, W), s.dtype),
            jax.ShapeDtypeStruct((E, W), s.dtype),
        ],
        mesh=_sc_mesh(),
        scratch_types=[
            pltpu.VMEM((ngrp, GB), jnp.int32),
            pltpu.VMEM((ngrp, GB), jnp.int32),
            pltpu.VMEM((GB, W), s.dtype),
            pltpu.VMEM((GB, W), s.dtype),
            pltpu.VMEM((GB, W), s.dtype),
            pltpu.VMEM((GB, W), s.dtype),
            pltpu.SemaphoreType.DMA,
            pltpu.SemaphoreType.DMA,
            pltpu.SemaphoreType.DMA,
            pltpu.SemaphoreType.DMA,
            pltpu.SemaphoreType.DMA,
        ],
    )
    def k(s_hbm, d_hbm, src_hbm, dst_hbm, sg_hbm, dg_hbm, sidx, didx,
          sa, da, sb, db, fs1, fs2, fs3, fs4, wsem):
        cid = lax.axis_index("c")
        sid = lax.axis_index("s")
        tid = cid * NSUB + sid

        pltpu.sync_copy(src_hbm.at[tid], sidx)
        pltpu.sync_copy(dst_hbm.at[tid], didx)
        base_e = tid * ept

        def body2(i, _):
            g0 = 2 * i
            fsa = pltpu.async_copy(s_hbm.at[sidx.at[g0]], sa, fs1)
            fda = pltpu.async_copy(d_hbm.at[didx.at[g0]], da, fs2)
            fsb = pltpu.async_copy(s_hbm.at[sidx.at[g0 + 1]], sb, fs3)
            fdb = pltpu.async_copy(d_hbm.at[didx.at[g0 + 1]], db, fs4)
            fsa.wait()
            w1 = pltpu.async_copy(sa, sg_hbm.at[pl.ds(base_e + g0 * GB, GB)],
                                  wsem)
            fda.wait()
            w2 = pltpu.async_copy(da, dg_hbm.at[pl.ds(base_e + g0 * GB, GB)],
                                  wsem)
            fsb.wait()
            w3 = pltpu.async_copy(
                sb, sg_hbm.at[pl.ds(base_e + (g0 + 1) * GB, GB)], wsem)
            fdb.wait()
            w4 = pltpu.async_copy(
                db, dg_hbm.at[pl.ds(base_e + (g0 + 1) * GB, GB)], wsem)
            w1.wait()
            w2.wait()
            w3.wait()
            w4.wait()
            return 0
        lax.fori_loop(0, ngrp // 2, body2, 0)

        g = ngrp - 1
        cps = pltpu.async_copy(s_hbm.at[sidx.at[g]], sa, fs1)
        cpd = pltpu.async_copy(d_hbm.at[didx.at[g]], da, fs2)
        cps.wait()
        pltpu.sync_copy(sa, sg_hbm.at[pl.ds(base_e + g * GB, GB)])
        cpd.wait()
        pltpu.sync_copy(da, dg_hbm.at[pl.ds(base_e + g * GB, GB)])

    return k(s, d, src3, dst3)


def kernel(h_V, edge_index, h_E, batch_id, W1, b1, W2, b2, Wr, br, W11, b11,
           W12, b12, Wg1, bg1, Wg2, bg2, g0, be0, g1, be1, gbn, bbn):
    N = h_V.shape[0]
    E = h_E.shape[0]
    blk_e = 2000
    nblk = _node_block(N)
    nh = -(-N // (2 * NSUB * GB)) * GB * NSUB

    src_w = edge_index[0].reshape(NW, E // (NW * GB), GB)
    dst_w = edge_index[1].reshape(NW, E // (NW * GB), GB)
    dst_s = edge_index[1].reshape(NSUB, E // (NSUB * GB), GB)
    bid3 = batch_id.reshape(N // nblk, 1, nblk)

    r = lambda v: v.reshape(1, H)
    b1r, b2r, brr, b11r, b12r = r(b1), r(b2), r(br), r(b11), r(b12)
    bg1r, bg2r = r(bg1), r(bg2)
    g0r, be0r, g1r, be1r, gbnr, bbnr = r(g0), r(be0), r(g1), r(be1), r(gbn), r(bbn)
    W11a = W11[0:H]
    W11b = W11[H:2 * H]
    W11c = W11[2 * H:3 * H]

    xg1 = _gather_one(h_V, src_w, E)
    w, msg1 = _edge_nn_mul(h_E, xg1, W1, b1r, W2, b2r, blk_e)
    agg1 = _scatter_add(msg1, dst_s, nh).reshape(2 * nh, H)
    h1 = _node_update(h_V, h_V, agg1[:N], Wr, brr, g0r, be0r)

    xg2 = _gather_one(h1, src_w, E)
    msg2 = _mul(xg2, w, blk_e)
    agg2 = _scatter_add(msg2, dst_s, nh).reshape(2 * nh, H)
    h2, s, d = _node_update2(h_V, h1, agg2[:N], Wr, brr, g1r, be1r, W11a, W11c)

    sg, dg = _gather_pair(s, d, src_w, dst_w, E)
    h_E_out = _edge_mlp(sg, dg, h_E, W11b, b11r, W12, b12r, gbnr, bbnr, blk_e)

    gate = _context_gate(h2, bid3, Wg1, bg1r, Wg2, bg2r)
    h_V_out = _apply_gate(h2, bid3, gate)

    return (h_V_out, h_E_out)
